# baseline (speedup 1.0000x reference)
import functools

import jax
import jax.numpy as jnp
from jax import lax
from jax.experimental import pallas as pl
from jax.experimental.pallas import tpu as pltpu
from jax.experimental.pallas import tpu_sc as plsc

_NC = 2
_NS = 16


@functools.lru_cache(maxsize=None)
def _build_native(v: int, d: int, ni: int, nj: int):
    ic = ni // _NS
    kpc = d // _NC
    ng = ic // 128
    mesh = plsc.VectorSubcoreMesh(core_axis_name="c", subcore_axis_name="s")

    @functools.partial(
        pl.kernel,
        mesh=mesh,
        compiler_params=pltpu.CompilerParams(use_tc_tiling_on_sc=False),
        out_type=jax.ShapeDtypeStruct((nj, d, ni), jnp.float32),
        scratch_types=[
            pltpu.VMEM_SHARED((v,), jnp.float32),
            pltpu.VMEM((nj, ic), jnp.int32),
            pltpu.VMEM((2, ic), jnp.float32),
            pltpu.SemaphoreType.DMA,
            pltpu.SemaphoreType.DMA,
            pltpu.SemaphoreType.DMA,
            pltpu.SemaphoreType.DMA,
            pltpu.SemaphoreType.DMA,
        ],
    )
    def k(tT_hbm, iT_hbm, out_hbm, row_sp, idx_v, slab, sem_r,
          sem_g0, sem_g1, sem_w0, sem_w1):
        sc = lax.axis_index("c")
        t = lax.axis_index("s")
        i0 = t * ic
        k_base = sc * kpc

        for j in range(nj):
            pltpu.sync_copy(iT_hbm.at[j, pl.ds(i0, ic)], idx_v.at[j])

        @pl.when(t == 0)
        def _():
            pltpu.async_copy(tT_hbm.at[k_base], row_sp, sem_r)

        sem_gs = (sem_g0, sem_g1)
        sem_ws = (sem_w0, sem_w1)

        def gather_to(j, b2):
            for c in range(ng):
                pltpu.async_copy(
                    row_sp.at[idx_v.at[j, pl.ds(c * 128, 128)]],
                    slab.at[b2, pl.ds(c * 128, 128)],
                    sem_gs[b2],
                )

        def drain_gather(b2):
            for c in range(ng):
                pltpu.make_async_copy(
                    tT_hbm.at[k_base, pl.ds(0, 128)],
                    slab.at[b2, pl.ds(c * 128, 128)],
                    sem_gs[b2],
                ).wait()

        def drain_write(b2):
            pltpu.make_async_copy(
                tT_hbm.at[k_base, pl.ds(0, ic)], slab.at[0], sem_ws[b2]
            ).wait()

        def do_feature(kk, carry):
            @pl.when(t == 0)
            def _():
                pltpu.make_async_copy(
                    tT_hbm.at[k_base], row_sp, sem_r
                ).wait()
            plsc.subcore_barrier()

            gather_to(0, 0)
            gather_to(1, 1)

            def jstep(s, carry2):
                j = 2 * s
                for b2 in range(2):
                    jj = j + b2
                    drain_gather(b2)
                    pltpu.async_copy(
                        slab.at[b2],
                        out_hbm.at[jj, k_base + kk, pl.ds(i0, ic)],
                        sem_ws[b2],
                    )

                    @pl.when(jj + 2 < nj)
                    def _():
                        drain_write(b2)
                        gather_to(jj + 2, b2)

                return carry2

            lax.fori_loop(0, nj // 2, jstep, 0)
            drain_write(0)
            drain_write(1)
            plsc.subcore_barrier()

            @pl.when(jnp.logical_and(t == 0, kk + 1 < kpc))
            def _():
                pltpu.async_copy(tT_hbm.at[k_base + kk + 1], row_sp, sem_r)

            return carry

        lax.fori_loop(0, kpc, do_feature, 0)

    return k


def kernel(indices, embedding):
    v, d = embedding.shape
    ni, nj = indices.shape
    tT = embedding.T
    iT = indices.T.astype(jnp.int32)
    outT = _build_native(v, d, ni, nj)(tT, iT)
    return outT.transpose(2, 0, 1)


# speedup vs baseline: 5.8466x; 5.8466x over previous
import functools

import jax
import jax.numpy as jnp
from jax import lax
from jax.experimental import pallas as pl
from jax.experimental.pallas import tpu as pltpu
from jax.experimental.pallas import tpu_sc as plsc

_NC = 2
_NS = 16
_NW = _NC * _NS
_CHUNK = 128
_KC = 4
_NBUF = 2


@functools.lru_cache(maxsize=None)
def _build_gather(n_chunks: int, d: int):
    b_per_w = n_chunks * _CHUNK
    n_outer = n_chunks // _KC
    mesh = plsc.VectorSubcoreMesh(core_axis_name="c", subcore_axis_name="s")

    @functools.partial(
        pl.kernel,
        mesh=mesh,
        compiler_params=pltpu.CompilerParams(use_tc_tiling_on_sc=False),
        out_type=jax.ShapeDtypeStruct((_NW * n_chunks, _CHUNK, d), jnp.float32),
        scratch_types=[
            pltpu.VMEM((n_chunks, _CHUNK), jnp.int32),
            pltpu.VMEM((_KC, _CHUNK, d), jnp.float32),
            pltpu.VMEM((_KC, _CHUNK, d), jnp.float32),
            pltpu.SemaphoreType.DMA,
            pltpu.SemaphoreType.DMA,
        ],
    )
    def k(table_hbm, idx_hbm, out_hbm, idx_v, buf0, buf1, sem0, sem1):
        wid = lax.axis_index("s") * _NC + lax.axis_index("c")
        base = wid * n_chunks
        pltpu.sync_copy(idx_hbm.at[wid], idx_v)

        bufs = (buf0, buf1)
        sems = (sem0, sem1)

        def fire(jj, b):
            for kk in range(_KC):
                pltpu.async_copy(
                    table_hbm.at[idx_v.at[jj * _KC + kk]],
                    bufs[b].at[kk],
                    sems[b],
                )

        def drain(b):
            for kk in range(_KC):
                pltpu.make_async_copy(
                    table_hbm.at[pl.ds(0, _CHUNK)], bufs[b].at[kk], sems[b]
                ).wait()

        for b in range(_NBUF):
            fire(b, b)

        def step(t, carry):
            j = t * _NBUF
            for b in range(_NBUF):
                jj = j + b
                drain(b)
                pltpu.sync_copy(
                    bufs[b], out_hbm.at[pl.ds(base + jj * _KC, _KC)]
                )
                fire(jj + _NBUF, b)
            return carry

        lax.fori_loop(0, n_outer // _NBUF - 1, step, 0)

        for b in range(_NBUF):
            jj = n_outer - _NBUF + b
            drain(b)
            pltpu.sync_copy(bufs[b], out_hbm.at[pl.ds(base + jj * _KC, _KC)])

    return k


def kernel(indices, embedding):
    d = embedding.shape[1]
    flat = indices.reshape(-1).astype(jnp.int32)
    b = flat.shape[0]
    grain = _NW * _CHUNK * _KC
    b_pad = ((b + grain - 1) // grain) * grain
    if b_pad != b:
        flat = jnp.pad(flat, (0, b_pad - b))
    n_chunks = b_pad // (_NW * _CHUNK)
    idx3 = flat.reshape(_NW, n_chunks, _CHUNK)
    out = _build_gather(n_chunks, d)(embedding, idx3)
    out = out.reshape(b_pad, d)
    if b_pad != b:
        out = out[:b]
    return out.reshape(indices.shape + (d,))
